# baseline (speedup 1.0000x reference)
import functools

import jax
import jax.numpy as jnp
from jax import lax
from jax.experimental import pallas as pl
from jax.experimental.pallas import tpu as pltpu
from jax.experimental.pallas import tpu_sc as plsc

B, S, H = 1, 2048, 768
E, K = 8, 2
INTER = 3072
LIMIT = 7.0
ALPHA = 1.702

N_FLAT = S * K
BM = 512
BN = 3072
CB = INTER // BN
NB = N_FLAT // BM + E
N_PAD = NB * BM

NUM_WORKERS = 32
GCHUNK = 64


def _router_body(hs_ref, rw_ref, rb_ref, idx_ref, w_ref, pack_ref):
    hs = hs_ref[...]
    hsb = hs.astype(jnp.bfloat16)
    lo = lax.bitcast_convert_type(hsb[:, :H // 2], jnp.uint16).astype(jnp.uint32)
    hi = lax.bitcast_convert_type(hsb[:, H // 2:], jnp.uint16).astype(jnp.uint32)
    pack_ref[...] = lax.bitcast_convert_type(lo | (hi << 16), jnp.int32)
    logits = jnp.dot(hs, rw_ref[...],
                     preferred_element_type=jnp.float32) + rb_ref[...]
    m1 = jnp.max(logits, axis=1)
    a1 = jnp.argmax(logits, axis=1).astype(jnp.int32)
    col = lax.broadcasted_iota(jnp.int32, (S, E), 1)
    masked = jnp.where(col == a1[:, None], -jnp.inf, logits)
    m2 = jnp.max(masked, axis=1)
    a2 = jnp.argmax(masked, axis=1).astype(jnp.int32)
    w1 = 1.0 / (1.0 + jnp.exp(m2 - m1))
    w2 = 1.0 - w1
    idx_ref[...] = jnp.where(col == 0, a1[:, None],
                             jnp.where(col == 1, a2[:, None], 0))
    w_ref[...] = jnp.where(col == 0, w1[:, None],
                           jnp.where(col == 1, w2[:, None], 0.0))


def _router(hs2d, router_weight, router_bias):
    return pl.pallas_call(
        _router_body,
        out_shape=(jax.ShapeDtypeStruct((S, E), jnp.int32),
                   jax.ShapeDtypeStruct((S, E), jnp.float32),
                   jax.ShapeDtypeStruct((S, H // 2), jnp.int32)),
    )(hs2d, router_weight, router_bias.reshape(1, E))


def _cast_body(src_ref, dst_ref):
    dst_ref[...] = src_ref[...].astype(jnp.bfloat16)


def _cast_w(w, bj):
    e, k, n = w.shape
    grid = (e, n // bj)
    return pl.pallas_call(
        _cast_body,
        grid=grid,
        in_specs=[pl.BlockSpec((1, k, bj), lambda i, j: (i, 0, j))],
        out_specs=pl.BlockSpec((1, k, bj), lambda i, j: (i, 0, j)),
        out_shape=jax.ShapeDtypeStruct(w.shape, jnp.bfloat16),
    )(w)


TOK_PER_W = S // NUM_WORKERS


def _dispatch_body(hs_hbm, pos3_hbm, w16_hbm, out_hbm, wout_hbm,
                   idx_v, rows_v, w0_v, w1_v, sem, wsem):
    wid = lax.axis_index("s") * 2 + lax.axis_index("c")
    base = wid * TOK_PER_W
    pltpu.sync_copy(pos3_hbm.at[wid], idx_v)
    pltpu.sync_copy(hs_hbm.at[pl.ds(base, TOK_PER_W)], rows_v)
    pltpu.sync_copy(w16_hbm.at[0, pl.ds(base, TOK_PER_W)], w0_v)
    pltpu.sync_copy(w16_hbm.at[1, pl.ds(base, TOK_PER_W)], w1_v)
    s0 = pltpu.async_copy(rows_v, out_hbm.at[idx_v.at[0]], sem)
    s1 = pltpu.async_copy(rows_v, out_hbm.at[idx_v.at[1]], sem)
    t0 = pltpu.async_copy(w0_v, wout_hbm.at[idx_v.at[0]], wsem)
    t1 = pltpu.async_copy(w1_v, wout_hbm.at[idx_v.at[1]], wsem)
    s0.wait()
    s1.wait()
    t0.wait()
    t1.wait()


@functools.cache
def _make_dispatch():
    return functools.partial(
        pl.kernel,
        mesh=plsc.VectorSubcoreMesh(core_axis_name="c", subcore_axis_name="s"),
        out_type=(jax.ShapeDtypeStruct((N_PAD, H // 2), jnp.int32),
                  jax.ShapeDtypeStruct((N_PAD, 128), jnp.float32)),
        scratch_types=[
            pltpu.VMEM((K, TOK_PER_W), jnp.int32),
            pltpu.VMEM((TOK_PER_W, H // 2), jnp.int32),
            pltpu.VMEM((TOK_PER_W, 128), jnp.float32),
            pltpu.VMEM((TOK_PER_W, 128), jnp.float32),
            pltpu.SemaphoreType.DMA,
            pltpu.SemaphoreType.DMA,
        ],
    )(_dispatch_body)


def _dispatch(hs_pack, pos3, w16):
    return _make_dispatch()(hs_pack, pos3, w16)


def _gateup_body(eob_ref, x_ref, wg_ref, wu_ref, b_ref, act_ref):
    cb = pl.program_id(0)
    xu = lax.bitcast_convert_type(x_ref[...], jnp.uint32)
    lo = lax.bitcast_convert_type(
        (xu & 0xFFFF).astype(jnp.uint16), jnp.bfloat16)
    hi = lax.bitcast_convert_type(
        (xu >> 16).astype(jnp.uint16), jnp.bfloat16)
    x = jnp.concatenate([lo, hi], axis=1)
    bg = b_ref[0, :, pl.ds(cb * BN, BN)]
    bu = b_ref[0, :, pl.ds(INTER + cb * BN, BN)]
    gate = jnp.dot(x, wg_ref[0], preferred_element_type=jnp.float32) + bg
    up = jnp.dot(x, wu_ref[0], preferred_element_type=jnp.float32) + bu
    gate = jnp.minimum(gate, LIMIT)
    up = jnp.clip(up, -LIMIT, LIMIT)
    glu = gate * (1.0 / (1.0 + jnp.exp(-ALPHA * gate)))
    act_ref[...] = ((up + 1.0) * glu).astype(jnp.bfloat16)


def _gateup(eob, x_sorted, gate_up_proj, gate_up_proj_bias):
    grid = (CB, NB)
    return pl.pallas_call(
        _gateup_body,
        grid_spec=pltpu.PrefetchScalarGridSpec(
            num_scalar_prefetch=1,
            grid=grid,
            in_specs=[
                pl.BlockSpec((BM, H // 2), lambda cb, rb, eob: (rb, 0)),
                pl.BlockSpec((1, H, BN), lambda cb, rb, eob: (eob[rb], 0, cb)),
                pl.BlockSpec((1, H, BN), lambda cb, rb, eob: (eob[rb], 0, CB + cb)),
                pl.BlockSpec((1, 1, 2 * INTER), lambda cb, rb, eob: (eob[rb], 0, 0)),
            ],
            out_specs=pl.BlockSpec((BM, BN), lambda cb, rb, eob: (rb, cb)),
        ),
        out_shape=jax.ShapeDtypeStruct((N_PAD, INTER), jnp.bfloat16),
    )(eob, x_sorted, gate_up_proj, gate_up_proj,
      gate_up_proj_bias.reshape(E, 1, 2 * INTER))


def _down_body(eob_ref, act_ref, wd_ref, bd_ref, w_ref, out_ref):
    y = jnp.dot(act_ref[...], wd_ref[0],
                preferred_element_type=jnp.float32) + bd_ref[0]
    out_ref[...] = y * w_ref[:, :1]


def _down(eob, act, down_proj, down_proj_bias, w_rows):
    grid = (NB,)
    return pl.pallas_call(
        _down_body,
        grid_spec=pltpu.PrefetchScalarGridSpec(
            num_scalar_prefetch=1,
            grid=grid,
            in_specs=[
                pl.BlockSpec((BM, INTER), lambda rb, eob: (rb, 0)),
                pl.BlockSpec((1, INTER, H), lambda rb, eob: (eob[rb], 0, 0)),
                pl.BlockSpec((1, 1, H), lambda rb, eob: (eob[rb], 0, 0)),
                pl.BlockSpec((BM, 128), lambda rb, eob: (rb, 0)),
            ],
            out_specs=pl.BlockSpec((BM, H), lambda rb, eob: (rb, 0)),
        ),
        out_shape=jax.ShapeDtypeStruct((N_PAD, H), jnp.float32),
    )(eob, act, down_proj, down_proj_bias.reshape(E, 1, H), w_rows)


TOK_PER_W = S // NUM_WORKERS


def _combine_body(rows_hbm, p0_hbm, p1_hbm, out_hbm, i0_v, i1_v, a_v, b_v, sem):
    wid = lax.axis_index("s") * 2 + lax.axis_index("c")
    base = wid * TOK_PER_W
    pltpu.sync_copy(p0_hbm.at[pl.ds(base, TOK_PER_W)], i0_v)
    pltpu.sync_copy(p1_hbm.at[pl.ds(base, TOK_PER_W)], i1_v)
    pltpu.async_copy(rows_hbm.at[i0_v], a_v, sem).wait()
    pltpu.async_copy(rows_hbm.at[i1_v], b_v, sem).wait()

    def add_row(i, carry):
        for j in range(H // 16):
            sl = pl.ds(j * 16, 16)
            a_v[i, sl] += b_v[i, sl]
        return carry

    lax.fori_loop(0, TOK_PER_W, add_row, 0)
    pltpu.sync_copy(a_v, out_hbm.at[pl.ds(base, TOK_PER_W)])


@functools.cache
def _make_combine():
    return functools.partial(
        pl.kernel,
        mesh=plsc.VectorSubcoreMesh(core_axis_name="c", subcore_axis_name="s"),
        out_type=jax.ShapeDtypeStruct((S, H), jnp.float32),
        scratch_types=[
            pltpu.VMEM((TOK_PER_W,), jnp.int32),
            pltpu.VMEM((TOK_PER_W,), jnp.int32),
            pltpu.VMEM((TOK_PER_W, H), jnp.float32),
            pltpu.VMEM((TOK_PER_W, H), jnp.float32),
            pltpu.SemaphoreType.DMA,
        ],
    )(_combine_body)


def _combine(rows, p0, p1):
    return _make_combine()(rows, p0, p1)


def kernel(hidden_states, router_weight, router_bias, gate_up_proj,
           gate_up_proj_bias, down_proj, down_proj_bias):
    hs2d = hidden_states.reshape(S, H)

    idx_out, w_out, hs_pack = _router(hs2d, router_weight, router_bias)
    top_idx = idx_out[:, :K]
    w_flat = w_out[:, :K].reshape(-1)
    e_flat = top_idx.reshape(-1)

    oh = (e_flat[:, None] == jnp.arange(E, dtype=jnp.int32)[None, :])
    csum = jnp.cumsum(oh.astype(jnp.int32), axis=0)
    g = csum[-1]
    rank = jnp.sum(jnp.where(oh, csum, 0), axis=1) - 1
    nb = (g + BM - 1) // BM
    cum_nb = jnp.cumsum(nb)
    b_ids = jnp.arange(NB, dtype=jnp.int32)
    eob = jnp.minimum(
        jnp.sum(b_ids[:, None] >= cum_nb[None, :], axis=1), E - 1
    ).astype(jnp.int32)
    po = jnp.concatenate([jnp.zeros(1, jnp.int32),
                          (jnp.cumsum(nb * BM)[:-1]).astype(jnp.int32)])

    pos = (jnp.sum(jnp.where(oh, po[None, :], 0), axis=1)
           + rank).astype(jnp.int32)
    pos2 = pos.reshape(S, K)
    p0 = pos2[:, 0]
    p1 = pos2[:, 1]
    pos3 = pos2.reshape(NUM_WORKERS, TOK_PER_W, K).transpose(0, 2, 1)
    w16 = jnp.pad(w_out[:, :K].T.reshape(K, S, 1), ((0, 0), (0, 0), (0, 127)))

    x_sorted, w_rows = _dispatch(hs_pack, pos3, w16)
    act = _gateup(eob, x_sorted, _cast_w(gate_up_proj, 2 * INTER // 8),
                  gate_up_proj_bias)
    rows = _down(eob, act, _cast_w(down_proj, H // 2), down_proj_bias,
                 w_rows)
    out = _combine(rows, p0, p1)
    return out.reshape(B, S, H)


# speedup vs baseline: 1.0272x; 1.0272x over previous
import functools

import jax
import jax.numpy as jnp
from jax import lax
from jax.experimental import pallas as pl
from jax.experimental.pallas import tpu as pltpu
from jax.experimental.pallas import tpu_sc as plsc

B, S, H = 1, 2048, 768
E, K = 8, 2
INTER = 3072
LIMIT = 7.0
ALPHA = 1.702

N_FLAT = S * K
BM = 256
BN = 3072
CB = INTER // BN
NB = N_FLAT // BM + E
N_PAD = NB * BM

NUM_WORKERS = 32
GCHUNK = 64


def _router_body(hs_ref, rw_ref, rb_ref, idx_ref, w_ref, pack_ref):
    hs = hs_ref[...]
    hsb = hs.astype(jnp.bfloat16)
    lo = lax.bitcast_convert_type(hsb[:, :H // 2], jnp.uint16).astype(jnp.uint32)
    hi = lax.bitcast_convert_type(hsb[:, H // 2:], jnp.uint16).astype(jnp.uint32)
    pack_ref[...] = lax.bitcast_convert_type(lo | (hi << 16), jnp.int32)
    logits = jnp.dot(hs, rw_ref[...],
                     preferred_element_type=jnp.float32) + rb_ref[...]
    m1 = jnp.max(logits, axis=1)
    a1 = jnp.argmax(logits, axis=1).astype(jnp.int32)
    col = lax.broadcasted_iota(jnp.int32, (S, E), 1)
    masked = jnp.where(col == a1[:, None], -jnp.inf, logits)
    m2 = jnp.max(masked, axis=1)
    a2 = jnp.argmax(masked, axis=1).astype(jnp.int32)
    w1 = 1.0 / (1.0 + jnp.exp(m2 - m1))
    w2 = 1.0 - w1
    idx_ref[...] = jnp.where(col == 0, a1[:, None],
                             jnp.where(col == 1, a2[:, None], 0))
    w_ref[...] = jnp.where(col == 0, w1[:, None],
                           jnp.where(col == 1, w2[:, None], 0.0))


def _router(hs2d, router_weight, router_bias):
    return pl.pallas_call(
        _router_body,
        out_shape=(jax.ShapeDtypeStruct((S, E), jnp.int32),
                   jax.ShapeDtypeStruct((S, E), jnp.float32),
                   jax.ShapeDtypeStruct((S, H // 2), jnp.int32)),
    )(hs2d, router_weight, router_bias.reshape(1, E))


def _cast_body(src_ref, dst_ref):
    dst_ref[...] = src_ref[...].astype(jnp.bfloat16)


def _cast_w(w, bj):
    e, k, n = w.shape
    grid = (e, n // bj)
    return pl.pallas_call(
        _cast_body,
        grid=grid,
        in_specs=[pl.BlockSpec((1, k, bj), lambda i, j: (i, 0, j))],
        out_specs=pl.BlockSpec((1, k, bj), lambda i, j: (i, 0, j)),
        out_shape=jax.ShapeDtypeStruct(w.shape, jnp.bfloat16),
    )(w)


TOK_PER_W = S // NUM_WORKERS


def _dispatch_body(hs_hbm, pos3_hbm, w16_hbm, out_hbm, wout_hbm,
                   idx_v, rows_v, w0_v, w1_v, sem, wsem):
    wid = lax.axis_index("s") * 2 + lax.axis_index("c")
    base = wid * TOK_PER_W
    pltpu.sync_copy(pos3_hbm.at[wid], idx_v)
    pltpu.sync_copy(hs_hbm.at[pl.ds(base, TOK_PER_W)], rows_v)
    pltpu.sync_copy(w16_hbm.at[0, pl.ds(base, TOK_PER_W)], w0_v)
    pltpu.sync_copy(w16_hbm.at[1, pl.ds(base, TOK_PER_W)], w1_v)
    s0 = pltpu.async_copy(rows_v, out_hbm.at[idx_v.at[0]], sem)
    s1 = pltpu.async_copy(rows_v, out_hbm.at[idx_v.at[1]], sem)
    t0 = pltpu.async_copy(w0_v, wout_hbm.at[idx_v.at[0]], wsem)
    t1 = pltpu.async_copy(w1_v, wout_hbm.at[idx_v.at[1]], wsem)
    s0.wait()
    s1.wait()
    t0.wait()
    t1.wait()


@functools.cache
def _make_dispatch():
    return functools.partial(
        pl.kernel,
        mesh=plsc.VectorSubcoreMesh(core_axis_name="c", subcore_axis_name="s"),
        out_type=(jax.ShapeDtypeStruct((N_PAD, H // 2), jnp.int32),
                  jax.ShapeDtypeStruct((N_PAD, 128), jnp.float32)),
        scratch_types=[
            pltpu.VMEM((K, TOK_PER_W), jnp.int32),
            pltpu.VMEM((TOK_PER_W, H // 2), jnp.int32),
            pltpu.VMEM((TOK_PER_W, 128), jnp.float32),
            pltpu.VMEM((TOK_PER_W, 128), jnp.float32),
            pltpu.SemaphoreType.DMA,
            pltpu.SemaphoreType.DMA,
        ],
    )(_dispatch_body)


def _dispatch(hs_pack, pos3, w16):
    return _make_dispatch()(hs_pack, pos3, w16)


def _gateup_body(eob_ref, x_ref, wg_ref, wu_ref, b_ref, act_ref):
    cb = pl.program_id(0)
    xu = lax.bitcast_convert_type(x_ref[...], jnp.uint32)
    lo = lax.bitcast_convert_type(
        (xu & 0xFFFF).astype(jnp.uint16), jnp.bfloat16)
    hi = lax.bitcast_convert_type(
        (xu >> 16).astype(jnp.uint16), jnp.bfloat16)
    x = jnp.concatenate([lo, hi], axis=1)
    bg = b_ref[0, :, pl.ds(cb * BN, BN)]
    bu = b_ref[0, :, pl.ds(INTER + cb * BN, BN)]
    gate = jnp.dot(x, wg_ref[0], preferred_element_type=jnp.float32) + bg
    up = jnp.dot(x, wu_ref[0], preferred_element_type=jnp.float32) + bu
    gate = jnp.minimum(gate, LIMIT)
    up = jnp.clip(up, -LIMIT, LIMIT)
    glu = gate * (1.0 / (1.0 + jnp.exp(-ALPHA * gate)))
    act_ref[...] = ((up + 1.0) * glu).astype(jnp.bfloat16)


def _gateup(eob, x_sorted, gate_up_proj, gate_up_proj_bias):
    grid = (CB, NB)
    return pl.pallas_call(
        _gateup_body,
        grid_spec=pltpu.PrefetchScalarGridSpec(
            num_scalar_prefetch=1,
            grid=grid,
            in_specs=[
                pl.BlockSpec((BM, H // 2), lambda cb, rb, eob: (rb, 0)),
                pl.BlockSpec((1, H, BN), lambda cb, rb, eob: (eob[rb], 0, cb)),
                pl.BlockSpec((1, H, BN), lambda cb, rb, eob: (eob[rb], 0, CB + cb)),
                pl.BlockSpec((1, 1, 2 * INTER), lambda cb, rb, eob: (eob[rb], 0, 0)),
            ],
            out_specs=pl.BlockSpec((BM, BN), lambda cb, rb, eob: (rb, cb)),
        ),
        out_shape=jax.ShapeDtypeStruct((N_PAD, INTER), jnp.bfloat16),
    )(eob, x_sorted, gate_up_proj, gate_up_proj,
      gate_up_proj_bias.reshape(E, 1, 2 * INTER))


def _down_body(eob_ref, act_ref, wd_ref, bd_ref, w_ref, out_ref):
    y = jnp.dot(act_ref[...], wd_ref[0],
                preferred_element_type=jnp.float32) + bd_ref[0]
    out_ref[...] = y * w_ref[:, :1]


def _down(eob, act, down_proj, down_proj_bias, w_rows):
    grid = (NB,)
    return pl.pallas_call(
        _down_body,
        grid_spec=pltpu.PrefetchScalarGridSpec(
            num_scalar_prefetch=1,
            grid=grid,
            in_specs=[
                pl.BlockSpec((BM, INTER), lambda rb, eob: (rb, 0)),
                pl.BlockSpec((1, INTER, H), lambda rb, eob: (eob[rb], 0, 0)),
                pl.BlockSpec((1, 1, H), lambda rb, eob: (eob[rb], 0, 0)),
                pl.BlockSpec((BM, 128), lambda rb, eob: (rb, 0)),
            ],
            out_specs=pl.BlockSpec((BM, H), lambda rb, eob: (rb, 0)),
        ),
        out_shape=jax.ShapeDtypeStruct((N_PAD, H), jnp.float32),
    )(eob, act, down_proj, down_proj_bias.reshape(E, 1, H), w_rows)


TOK_PER_W = S // NUM_WORKERS


def _combine_body(rows_hbm, p0_hbm, p1_hbm, out_hbm, i0_v, i1_v, a_v, b_v, sem):
    wid = lax.axis_index("s") * 2 + lax.axis_index("c")
    base = wid * TOK_PER_W
    pltpu.sync_copy(p0_hbm.at[pl.ds(base, TOK_PER_W)], i0_v)
    pltpu.sync_copy(p1_hbm.at[pl.ds(base, TOK_PER_W)], i1_v)
    pltpu.async_copy(rows_hbm.at[i0_v], a_v, sem).wait()
    pltpu.async_copy(rows_hbm.at[i1_v], b_v, sem).wait()

    def add_row(i, carry):
        for j in range(H // 16):
            sl = pl.ds(j * 16, 16)
            a_v[i, sl] += b_v[i, sl]
        return carry

    lax.fori_loop(0, TOK_PER_W, add_row, 0)
    pltpu.sync_copy(a_v, out_hbm.at[pl.ds(base, TOK_PER_W)])


@functools.cache
def _make_combine():
    return functools.partial(
        pl.kernel,
        mesh=plsc.VectorSubcoreMesh(core_axis_name="c", subcore_axis_name="s"),
        out_type=jax.ShapeDtypeStruct((S, H), jnp.float32),
        scratch_types=[
            pltpu.VMEM((TOK_PER_W,), jnp.int32),
            pltpu.VMEM((TOK_PER_W,), jnp.int32),
            pltpu.VMEM((TOK_PER_W, H), jnp.float32),
            pltpu.VMEM((TOK_PER_W, H), jnp.float32),
            pltpu.SemaphoreType.DMA,
        ],
    )(_combine_body)


def _combine(rows, p0, p1):
    return _make_combine()(rows, p0, p1)


def kernel(hidden_states, router_weight, router_bias, gate_up_proj,
           gate_up_proj_bias, down_proj, down_proj_bias):
    hs2d = hidden_states.reshape(S, H)

    idx_out, w_out, hs_pack = _router(hs2d, router_weight, router_bias)
    top_idx = idx_out[:, :K]
    w_flat = w_out[:, :K].reshape(-1)
    e_flat = top_idx.reshape(-1)

    oh = (e_flat[:, None] == jnp.arange(E, dtype=jnp.int32)[None, :])
    csum = jnp.cumsum(oh.astype(jnp.int32), axis=0)
    g = csum[-1]
    rank = jnp.sum(jnp.where(oh, csum, 0), axis=1) - 1
    nb = (g + BM - 1) // BM
    cum_nb = jnp.cumsum(nb)
    b_ids = jnp.arange(NB, dtype=jnp.int32)
    eob = jnp.minimum(
        jnp.sum(b_ids[:, None] >= cum_nb[None, :], axis=1), E - 1
    ).astype(jnp.int32)
    po = jnp.concatenate([jnp.zeros(1, jnp.int32),
                          (jnp.cumsum(nb * BM)[:-1]).astype(jnp.int32)])

    pos = (jnp.sum(jnp.where(oh, po[None, :], 0), axis=1)
           + rank).astype(jnp.int32)
    pos2 = pos.reshape(S, K)
    p0 = pos2[:, 0]
    p1 = pos2[:, 1]
    pos3 = pos2.reshape(NUM_WORKERS, TOK_PER_W, K).transpose(0, 2, 1)
    w16 = jnp.pad(w_out[:, :K].T.reshape(K, S, 1), ((0, 0), (0, 0), (0, 127)))

    x_sorted, w_rows = _dispatch(hs_pack, pos3, w16)
    act = _gateup(eob, x_sorted, _cast_w(gate_up_proj, 2 * INTER // 8),
                  gate_up_proj_bias)
    rows = _down(eob, act, _cast_w(down_proj, H // 2), down_proj_bias,
                 w_rows)
    out = _combine(rows, p0, p1)
    return out.reshape(B, S, H)


# speedup vs baseline: 1.0714x; 1.0430x over previous
import functools

import jax
import jax.numpy as jnp
from jax import lax
from jax.experimental import pallas as pl
from jax.experimental.pallas import tpu as pltpu
from jax.experimental.pallas import tpu_sc as plsc

B, S, H = 1, 2048, 768
E, K = 8, 2
INTER = 3072
LIMIT = 7.0
ALPHA = 1.702

N_FLAT = S * K
BM = 256
BN = 3072
CB = INTER // BN
NB = N_FLAT // BM + E
N_PAD = NB * BM

NUM_WORKERS = 32
GCHUNK = 64


def _router_body(hs_ref, rw_ref, rb_ref, idx_ref, w_ref, pack_ref, w16_ref):
    hs = hs_ref[...]
    hsb = hs.astype(jnp.bfloat16)
    lo = lax.bitcast_convert_type(hsb[:, :H // 2], jnp.uint16).astype(jnp.uint32)
    hi = lax.bitcast_convert_type(hsb[:, H // 2:], jnp.uint16).astype(jnp.uint32)
    pack_ref[...] = lax.bitcast_convert_type(lo | (hi << 16), jnp.int32)
    logits = jnp.dot(hs, rw_ref[...],
                     preferred_element_type=jnp.float32) + rb_ref[...]
    m1 = jnp.max(logits, axis=1)
    a1 = jnp.argmax(logits, axis=1).astype(jnp.int32)
    col = lax.broadcasted_iota(jnp.int32, (S, E), 1)
    masked = jnp.where(col == a1[:, None], -jnp.inf, logits)
    m2 = jnp.max(masked, axis=1)
    a2 = jnp.argmax(masked, axis=1).astype(jnp.int32)
    w1 = 1.0 / (1.0 + jnp.exp(m2 - m1))
    w2 = 1.0 - w1
    idx_ref[...] = jnp.where(col == 0, a1[:, None],
                             jnp.where(col == 1, a2[:, None], 0))
    w_ref[...] = jnp.where(col == 0, w1[:, None],
                           jnp.where(col == 1, w2[:, None], 0.0))
    lane = lax.broadcasted_iota(jnp.int32, (K, S, 128), 2)
    slot = lax.broadcasted_iota(jnp.int32, (K, S, 128), 0)
    w16_ref[...] = jnp.where(
        lane == 0,
        jnp.where(slot == 0, w1[None, :, None], w2[None, :, None]), 0.0)


def _router(hs2d, router_weight, router_bias):
    return pl.pallas_call(
        _router_body,
        out_shape=(jax.ShapeDtypeStruct((S, E), jnp.int32),
                   jax.ShapeDtypeStruct((S, E), jnp.float32),
                   jax.ShapeDtypeStruct((S, H // 2), jnp.int32),
                   jax.ShapeDtypeStruct((K, S, 128), jnp.float32)),
    )(hs2d, router_weight, router_bias.reshape(1, E))


def _cast_body(src_ref, dst_ref):
    dst_ref[...] = src_ref[...].astype(jnp.bfloat16)


def _cast_w(w, bj):
    e, k, n = w.shape
    grid = (e, n // bj)
    return pl.pallas_call(
        _cast_body,
        grid=grid,
        in_specs=[pl.BlockSpec((1, k, bj), lambda i, j: (i, 0, j))],
        out_specs=pl.BlockSpec((1, k, bj), lambda i, j: (i, 0, j)),
        out_shape=jax.ShapeDtypeStruct(w.shape, jnp.bfloat16),
    )(w)


TOK_PER_W = S // NUM_WORKERS


def _dispatch_body(hs_hbm, pos3_hbm, w16_hbm, out_hbm, wout_hbm,
                   idx_v, rows_v, w0_v, w1_v, sem, wsem):
    wid = lax.axis_index("s") * 2 + lax.axis_index("c")
    base = wid * TOK_PER_W
    pltpu.sync_copy(pos3_hbm.at[wid], idx_v)
    pltpu.sync_copy(hs_hbm.at[pl.ds(base, TOK_PER_W)], rows_v)
    pltpu.sync_copy(w16_hbm.at[0, pl.ds(base, TOK_PER_W)], w0_v)
    pltpu.sync_copy(w16_hbm.at[1, pl.ds(base, TOK_PER_W)], w1_v)
    s0 = pltpu.async_copy(rows_v, out_hbm.at[idx_v.at[0]], sem)
    s1 = pltpu.async_copy(rows_v, out_hbm.at[idx_v.at[1]], sem)
    t0 = pltpu.async_copy(w0_v, wout_hbm.at[idx_v.at[0]], wsem)
    t1 = pltpu.async_copy(w1_v, wout_hbm.at[idx_v.at[1]], wsem)
    s0.wait()
    s1.wait()
    t0.wait()
    t1.wait()


@functools.cache
def _make_dispatch():
    return functools.partial(
        pl.kernel,
        mesh=plsc.VectorSubcoreMesh(core_axis_name="c", subcore_axis_name="s"),
        out_type=(jax.ShapeDtypeStruct((N_PAD, H // 2), jnp.int32),
                  jax.ShapeDtypeStruct((N_PAD, 128), jnp.float32)),
        scratch_types=[
            pltpu.VMEM((K, TOK_PER_W), jnp.int32),
            pltpu.VMEM((TOK_PER_W, H // 2), jnp.int32),
            pltpu.VMEM((TOK_PER_W, 128), jnp.float32),
            pltpu.VMEM((TOK_PER_W, 128), jnp.float32),
            pltpu.SemaphoreType.DMA,
            pltpu.SemaphoreType.DMA,
        ],
    )(_dispatch_body)


def _dispatch(hs_pack, pos3, w16):
    return _make_dispatch()(hs_pack, pos3, w16)


def _gateup_body(eob_ref, x_ref, wg_ref, wu_ref, b_ref, act_ref):
    cb = pl.program_id(0)
    xu = lax.bitcast_convert_type(x_ref[...], jnp.uint32)
    lo = lax.bitcast_convert_type(
        (xu & 0xFFFF).astype(jnp.uint16), jnp.bfloat16)
    hi = lax.bitcast_convert_type(
        (xu >> 16).astype(jnp.uint16), jnp.bfloat16)
    x = jnp.concatenate([lo, hi], axis=1)
    bg = b_ref[0, :, pl.ds(cb * BN, BN)]
    bu = b_ref[0, :, pl.ds(INTER + cb * BN, BN)]
    gate = jnp.dot(x, wg_ref[0], preferred_element_type=jnp.float32) + bg
    up = jnp.dot(x, wu_ref[0], preferred_element_type=jnp.float32) + bu
    gate = jnp.minimum(gate, LIMIT)
    up = jnp.clip(up, -LIMIT, LIMIT)
    glu = gate * (1.0 / (1.0 + jnp.exp(-ALPHA * gate)))
    act_ref[...] = ((up + 1.0) * glu).astype(jnp.bfloat16)


def _gateup(eob, x_sorted, gate_up_proj, gate_up_proj_bias):
    grid = (CB, NB)
    return pl.pallas_call(
        _gateup_body,
        grid_spec=pltpu.PrefetchScalarGridSpec(
            num_scalar_prefetch=1,
            grid=grid,
            in_specs=[
                pl.BlockSpec((BM, H // 2), lambda cb, rb, eob: (rb, 0)),
                pl.BlockSpec((1, H, BN), lambda cb, rb, eob: (eob[rb], 0, cb)),
                pl.BlockSpec((1, H, BN), lambda cb, rb, eob: (eob[rb], 0, CB + cb)),
                pl.BlockSpec((1, 1, 2 * INTER), lambda cb, rb, eob: (eob[rb], 0, 0)),
            ],
            out_specs=pl.BlockSpec((BM, BN), lambda cb, rb, eob: (rb, cb)),
        ),
        out_shape=jax.ShapeDtypeStruct((N_PAD, INTER), jnp.bfloat16),
    )(eob, x_sorted, gate_up_proj, gate_up_proj,
      gate_up_proj_bias.reshape(E, 1, 2 * INTER))


def _down_body(eob_ref, act_ref, wd_ref, bd_ref, w_ref, out_ref):
    y = jnp.dot(act_ref[...], wd_ref[0],
                preferred_element_type=jnp.float32) + bd_ref[0]
    out_ref[...] = y * w_ref[:, :1]


def _down(eob, act, down_proj, down_proj_bias, w_rows):
    grid = (NB,)
    return pl.pallas_call(
        _down_body,
        grid_spec=pltpu.PrefetchScalarGridSpec(
            num_scalar_prefetch=1,
            grid=grid,
            in_specs=[
                pl.BlockSpec((BM, INTER), lambda rb, eob: (rb, 0)),
                pl.BlockSpec((1, INTER, H), lambda rb, eob: (eob[rb], 0, 0)),
                pl.BlockSpec((1, 1, H), lambda rb, eob: (eob[rb], 0, 0)),
                pl.BlockSpec((BM, 128), lambda rb, eob: (rb, 0)),
            ],
            out_specs=pl.BlockSpec((BM, H), lambda rb, eob: (rb, 0)),
        ),
        out_shape=jax.ShapeDtypeStruct((N_PAD, H), jnp.float32),
    )(eob, act, down_proj, down_proj_bias.reshape(E, 1, H), w_rows)


TOK_PER_W = S // NUM_WORKERS


def _combine_body(rows_hbm, p0_hbm, p1_hbm, out_hbm, i0_v, i1_v, a_v, b_v, sem):
    wid = lax.axis_index("s") * 2 + lax.axis_index("c")
    base = wid * TOK_PER_W
    pltpu.sync_copy(p0_hbm.at[pl.ds(base, TOK_PER_W)], i0_v)
    pltpu.sync_copy(p1_hbm.at[pl.ds(base, TOK_PER_W)], i1_v)
    pltpu.async_copy(rows_hbm.at[i0_v], a_v, sem).wait()
    pltpu.async_copy(rows_hbm.at[i1_v], b_v, sem).wait()

    def add_row(i, carry):
        for j in range(H // 16):
            sl = pl.ds(j * 16, 16)
            a_v[i, sl] += b_v[i, sl]
        return carry

    lax.fori_loop(0, TOK_PER_W, add_row, 0)
    pltpu.sync_copy(a_v, out_hbm.at[pl.ds(base, TOK_PER_W)])


@functools.cache
def _make_combine():
    return functools.partial(
        pl.kernel,
        mesh=plsc.VectorSubcoreMesh(core_axis_name="c", subcore_axis_name="s"),
        out_type=jax.ShapeDtypeStruct((S, H), jnp.float32),
        scratch_types=[
            pltpu.VMEM((TOK_PER_W,), jnp.int32),
            pltpu.VMEM((TOK_PER_W,), jnp.int32),
            pltpu.VMEM((TOK_PER_W, H), jnp.float32),
            pltpu.VMEM((TOK_PER_W, H), jnp.float32),
            pltpu.SemaphoreType.DMA,
        ],
    )(_combine_body)


def _combine(rows, p0, p1):
    return _make_combine()(rows, p0, p1)


def kernel(hidden_states, router_weight, router_bias, gate_up_proj,
           gate_up_proj_bias, down_proj, down_proj_bias):
    hs2d = hidden_states.reshape(S, H)

    idx_out, w_out, hs_pack, w16 = _router(hs2d, router_weight, router_bias)
    top_idx = idx_out[:, :K]
    w_flat = w_out[:, :K].reshape(-1)
    e_flat = top_idx.reshape(-1)

    oh = (e_flat[:, None] == jnp.arange(E, dtype=jnp.int32)[None, :])
    csum = jnp.cumsum(oh.astype(jnp.int32), axis=0)
    g = csum[-1]
    rank = jnp.sum(jnp.where(oh, csum, 0), axis=1) - 1
    nb = (g + BM - 1) // BM
    cum_nb = jnp.cumsum(nb)
    b_ids = jnp.arange(NB, dtype=jnp.int32)
    eob = jnp.minimum(
        jnp.sum(b_ids[:, None] >= cum_nb[None, :], axis=1), E - 1
    ).astype(jnp.int32)
    po = jnp.concatenate([jnp.zeros(1, jnp.int32),
                          (jnp.cumsum(nb * BM)[:-1]).astype(jnp.int32)])

    pos = (jnp.sum(jnp.where(oh, po[None, :], 0), axis=1)
           + rank).astype(jnp.int32)
    pos2 = pos.reshape(S, K)
    p0 = pos2[:, 0]
    p1 = pos2[:, 1]
    pos3 = pos2.reshape(NUM_WORKERS, TOK_PER_W, K).transpose(0, 2, 1)

    x_sorted, w_rows = _dispatch(hs_pack, pos3, w16)
    act = _gateup(eob, x_sorted, _cast_w(gate_up_proj, 2 * INTER // 8),
                  gate_up_proj_bias)
    rows = _down(eob, act, _cast_w(down_proj, H // 2), down_proj_bias,
                 w_rows)
    out = _combine(rows, p0, p1)
    return out.reshape(B, S, H)


# speedup vs baseline: 1.1564x; 1.0794x over previous
import functools

import jax
import jax.numpy as jnp
from jax import lax
from jax.experimental import pallas as pl
from jax.experimental.pallas import tpu as pltpu
from jax.experimental.pallas import tpu_sc as plsc

B, S, H = 1, 2048, 768
E, K = 8, 2
INTER = 3072
LIMIT = 7.0
ALPHA = 1.702

N_FLAT = S * K
BM = 256
BN = 3072
CB = INTER // BN
NB = N_FLAT // BM + E
N_PAD = NB * BM

NUM_WORKERS = 32
GCHUNK = 64


def _router_body(hs_ref, rw_ref, rb_ref, idx_ref, w_ref, pack_ref, w16_ref):
    hs = hs_ref[...]
    hsb = hs.astype(jnp.bfloat16)
    lo = lax.bitcast_convert_type(hsb[:, :H // 2], jnp.uint16).astype(jnp.uint32)
    hi = lax.bitcast_convert_type(hsb[:, H // 2:], jnp.uint16).astype(jnp.uint32)
    pack_ref[...] = lax.bitcast_convert_type(lo | (hi << 16), jnp.int32)
    logits = jnp.dot(hs, rw_ref[...],
                     preferred_element_type=jnp.float32) + rb_ref[...]
    m1 = jnp.max(logits, axis=1)
    a1 = jnp.argmax(logits, axis=1).astype(jnp.int32)
    col = lax.broadcasted_iota(jnp.int32, (S, E), 1)
    masked = jnp.where(col == a1[:, None], -jnp.inf, logits)
    m2 = jnp.max(masked, axis=1)
    a2 = jnp.argmax(masked, axis=1).astype(jnp.int32)
    w1 = 1.0 / (1.0 + jnp.exp(m2 - m1))
    w2 = 1.0 - w1
    idx_ref[...] = jnp.where(col == 0, a1[:, None],
                             jnp.where(col == 1, a2[:, None], 0))
    w_ref[...] = jnp.where(col == 0, w1[:, None],
                           jnp.where(col == 1, w2[:, None], 0.0))
    lane = lax.broadcasted_iota(jnp.int32, (K, S, 128), 2)
    slot = lax.broadcasted_iota(jnp.int32, (K, S, 128), 0)
    w16_ref[...] = jnp.where(
        lane == 0,
        jnp.where(slot == 0, w1[None, :, None], w2[None, :, None]), 0.0)


def _router(hs2d, router_weight, router_bias):
    return pl.pallas_call(
        _router_body,
        out_shape=(jax.ShapeDtypeStruct((S, E), jnp.int32),
                   jax.ShapeDtypeStruct((S, E), jnp.float32),
                   jax.ShapeDtypeStruct((S, H // 2), jnp.int32),
                   jax.ShapeDtypeStruct((K, S, 128), jnp.float32)),
    )(hs2d, router_weight, router_bias.reshape(1, E))


def _cast_body(src_ref, dst_ref):
    dst_ref[...] = src_ref[...].astype(jnp.bfloat16)


def _cast_w(w, bj):
    e, k, n = w.shape
    grid = (e, n // bj)
    return pl.pallas_call(
        _cast_body,
        grid=grid,
        in_specs=[pl.BlockSpec((1, k, bj), lambda i, j: (i, 0, j))],
        out_specs=pl.BlockSpec((1, k, bj), lambda i, j: (i, 0, j)),
        out_shape=jax.ShapeDtypeStruct(w.shape, jnp.bfloat16),
    )(w)


TOK_PER_W = S // NUM_WORKERS


def _dispatch_body(hs_hbm, pos3_hbm, w16_hbm, out_hbm, wout_hbm,
                   idx_v, rows_v, w0_v, w1_v, sem, wsem):
    wid = lax.axis_index("s") * 2 + lax.axis_index("c")
    base = wid * TOK_PER_W
    pltpu.sync_copy(pos3_hbm.at[wid], idx_v)
    pltpu.sync_copy(hs_hbm.at[pl.ds(base, TOK_PER_W)], rows_v)
    pltpu.sync_copy(w16_hbm.at[0, pl.ds(base, TOK_PER_W)], w0_v)
    pltpu.sync_copy(w16_hbm.at[1, pl.ds(base, TOK_PER_W)], w1_v)
    s0 = pltpu.async_copy(rows_v, out_hbm.at[idx_v.at[0]], sem)
    s1 = pltpu.async_copy(rows_v, out_hbm.at[idx_v.at[1]], sem)
    t0 = pltpu.async_copy(w0_v, wout_hbm.at[idx_v.at[0]], wsem)
    t1 = pltpu.async_copy(w1_v, wout_hbm.at[idx_v.at[1]], wsem)
    s0.wait()
    s1.wait()
    t0.wait()
    t1.wait()


@functools.cache
def _make_dispatch():
    return functools.partial(
        pl.kernel,
        mesh=plsc.VectorSubcoreMesh(core_axis_name="c", subcore_axis_name="s"),
        out_type=(jax.ShapeDtypeStruct((N_PAD, H // 2), jnp.int32),
                  jax.ShapeDtypeStruct((N_PAD, 128), jnp.float32)),
        scratch_types=[
            pltpu.VMEM((K, TOK_PER_W), jnp.int32),
            pltpu.VMEM((TOK_PER_W, H // 2), jnp.int32),
            pltpu.VMEM((TOK_PER_W, 128), jnp.float32),
            pltpu.VMEM((TOK_PER_W, 128), jnp.float32),
            pltpu.SemaphoreType.DMA,
            pltpu.SemaphoreType.DMA,
        ],
    )(_dispatch_body)


def _dispatch(hs_pack, pos3, w16):
    return _make_dispatch()(hs_pack, pos3, w16)


def _gateup_body(eob_ref, x_ref, wg_ref, wu_ref, b_ref, act_ref):
    cb = pl.program_id(0)
    xu = lax.bitcast_convert_type(x_ref[...], jnp.uint32)
    lo = lax.bitcast_convert_type(
        (xu & 0xFFFF).astype(jnp.uint16), jnp.bfloat16)
    hi = lax.bitcast_convert_type(
        (xu >> 16).astype(jnp.uint16), jnp.bfloat16)
    x = jnp.concatenate([lo, hi], axis=1)
    bg = b_ref[0, :, pl.ds(cb * BN, BN)]
    bu = b_ref[0, :, pl.ds(INTER + cb * BN, BN)]
    gate = jnp.dot(x, wg_ref[0], preferred_element_type=jnp.float32) + bg
    up = jnp.dot(x, wu_ref[0], preferred_element_type=jnp.float32) + bu
    gate = jnp.minimum(gate, LIMIT)
    up = jnp.clip(up, -LIMIT, LIMIT)
    glu = gate * (1.0 / (1.0 + jnp.exp(-ALPHA * gate)))
    act_ref[...] = ((up + 1.0) * glu).astype(jnp.bfloat16)


def _gateup(eob, x_sorted, gate_up_proj, gate_up_proj_bias):
    grid = (CB, NB)
    return pl.pallas_call(
        _gateup_body,
        grid_spec=pltpu.PrefetchScalarGridSpec(
            num_scalar_prefetch=1,
            grid=grid,
            in_specs=[
                pl.BlockSpec((BM, H // 2), lambda cb, rb, eob: (rb, 0)),
                pl.BlockSpec((1, H, BN), lambda cb, rb, eob: (eob[rb], 0, cb)),
                pl.BlockSpec((1, H, BN), lambda cb, rb, eob: (eob[rb], 0, CB + cb)),
                pl.BlockSpec((1, 1, 2 * INTER), lambda cb, rb, eob: (eob[rb], 0, 0)),
            ],
            out_specs=pl.BlockSpec((BM, BN), lambda cb, rb, eob: (rb, cb)),
        ),
        out_shape=jax.ShapeDtypeStruct((N_PAD, INTER), jnp.bfloat16),
    )(eob, x_sorted, gate_up_proj, gate_up_proj,
      gate_up_proj_bias.reshape(E, 1, 2 * INTER))


def _down_body(eob_ref, act_ref, wd_ref, bd_ref, w_ref, out_ref):
    y = jnp.dot(act_ref[...], wd_ref[0],
                preferred_element_type=jnp.float32) + bd_ref[0]
    out_ref[...] = y * w_ref[:, :1]


def _down(eob, act, down_proj, down_proj_bias, w_rows):
    grid = (NB,)
    return pl.pallas_call(
        _down_body,
        grid_spec=pltpu.PrefetchScalarGridSpec(
            num_scalar_prefetch=1,
            grid=grid,
            in_specs=[
                pl.BlockSpec((BM, INTER), lambda rb, eob: (rb, 0)),
                pl.BlockSpec((1, INTER, H), lambda rb, eob: (eob[rb], 0, 0)),
                pl.BlockSpec((1, 1, H), lambda rb, eob: (eob[rb], 0, 0)),
                pl.BlockSpec((BM, 128), lambda rb, eob: (rb, 0)),
            ],
            out_specs=pl.BlockSpec((BM, H), lambda rb, eob: (rb, 0)),
        ),
        out_shape=jax.ShapeDtypeStruct((N_PAD, H), jnp.float32),
    )(eob, act, down_proj, down_proj_bias.reshape(E, 1, H), w_rows)


def _mlp_body(eob_ref, x_ref, wgu_ref, b_ref, wd_ref, bd_ref, w_ref, out_ref):
    xu = lax.bitcast_convert_type(x_ref[...], jnp.uint32)
    lo = lax.bitcast_convert_type(
        (xu & 0xFFFF).astype(jnp.uint16), jnp.bfloat16)
    hi = lax.bitcast_convert_type(
        (xu >> 16).astype(jnp.uint16), jnp.bfloat16)
    x = jnp.concatenate([lo, hi], axis=1)
    gate = jnp.dot(x, wgu_ref[0, :, :INTER],
                   preferred_element_type=jnp.float32) + b_ref[0, :, :INTER]
    up = jnp.dot(x, wgu_ref[0, :, INTER:],
                 preferred_element_type=jnp.float32) + b_ref[0, :, INTER:]
    gate = jnp.minimum(gate, LIMIT)
    up = jnp.clip(up, -LIMIT, LIMIT)
    glu = gate * (1.0 / (1.0 + jnp.exp(-ALPHA * gate)))
    act = ((up + 1.0) * glu).astype(jnp.bfloat16)
    y = jnp.dot(act, wd_ref[0],
                preferred_element_type=jnp.float32) + bd_ref[0]
    out_ref[...] = y * w_ref[:, :1]


def _mlp(eob, x_sorted, wgu, gate_up_proj_bias, wd, down_proj_bias, w_rows):
    grid = (NB,)
    return pl.pallas_call(
        _mlp_body,
        grid_spec=pltpu.PrefetchScalarGridSpec(
            num_scalar_prefetch=1,
            grid=grid,
            in_specs=[
                pl.BlockSpec((BM, H // 2), lambda rb, eob: (rb, 0)),
                pl.BlockSpec((1, H, 2 * INTER), lambda rb, eob: (eob[rb], 0, 0)),
                pl.BlockSpec((1, 1, 2 * INTER), lambda rb, eob: (eob[rb], 0, 0)),
                pl.BlockSpec((1, INTER, H), lambda rb, eob: (eob[rb], 0, 0)),
                pl.BlockSpec((1, 1, H), lambda rb, eob: (eob[rb], 0, 0)),
                pl.BlockSpec((BM, 128), lambda rb, eob: (rb, 0)),
            ],
            out_specs=pl.BlockSpec((BM, H), lambda rb, eob: (rb, 0)),
        ),
        out_shape=jax.ShapeDtypeStruct((N_PAD, H), jnp.float32),
    )(eob, x_sorted, wgu, gate_up_proj_bias.reshape(E, 1, 2 * INTER),
      wd, down_proj_bias.reshape(E, 1, H), w_rows)


TOK_PER_W = S // NUM_WORKERS


def _combine_body(rows_hbm, p0_hbm, p1_hbm, out_hbm, i0_v, i1_v, a_v, b_v, sem):
    wid = lax.axis_index("s") * 2 + lax.axis_index("c")
    base = wid * TOK_PER_W
    pltpu.sync_copy(p0_hbm.at[pl.ds(base, TOK_PER_W)], i0_v)
    pltpu.sync_copy(p1_hbm.at[pl.ds(base, TOK_PER_W)], i1_v)
    pltpu.async_copy(rows_hbm.at[i0_v], a_v, sem).wait()
    pltpu.async_copy(rows_hbm.at[i1_v], b_v, sem).wait()

    def add_row(i, carry):
        for j in range(H // 16):
            sl = pl.ds(j * 16, 16)
            a_v[i, sl] += b_v[i, sl]
        return carry

    lax.fori_loop(0, TOK_PER_W, add_row, 0)
    pltpu.sync_copy(a_v, out_hbm.at[pl.ds(base, TOK_PER_W)])


@functools.cache
def _make_combine():
    return functools.partial(
        pl.kernel,
        mesh=plsc.VectorSubcoreMesh(core_axis_name="c", subcore_axis_name="s"),
        out_type=jax.ShapeDtypeStruct((S, H), jnp.float32),
        scratch_types=[
            pltpu.VMEM((TOK_PER_W,), jnp.int32),
            pltpu.VMEM((TOK_PER_W,), jnp.int32),
            pltpu.VMEM((TOK_PER_W, H), jnp.float32),
            pltpu.VMEM((TOK_PER_W, H), jnp.float32),
            pltpu.SemaphoreType.DMA,
        ],
    )(_combine_body)


def _combine(rows, p0, p1):
    return _make_combine()(rows, p0, p1)


def kernel(hidden_states, router_weight, router_bias, gate_up_proj,
           gate_up_proj_bias, down_proj, down_proj_bias):
    hs2d = hidden_states.reshape(S, H)

    idx_out, w_out, hs_pack, w16 = _router(hs2d, router_weight, router_bias)
    top_idx = idx_out[:, :K]
    w_flat = w_out[:, :K].reshape(-1)
    e_flat = top_idx.reshape(-1)

    oh = (e_flat[:, None] == jnp.arange(E, dtype=jnp.int32)[None, :])
    csum = jnp.cumsum(oh.astype(jnp.int32), axis=0)
    g = csum[-1]
    rank = jnp.sum(jnp.where(oh, csum, 0), axis=1) - 1
    nb = (g + BM - 1) // BM
    cum_nb = jnp.cumsum(nb)
    b_ids = jnp.arange(NB, dtype=jnp.int32)
    eob = jnp.minimum(
        jnp.sum(b_ids[:, None] >= cum_nb[None, :], axis=1), E - 1
    ).astype(jnp.int32)
    po = jnp.concatenate([jnp.zeros(1, jnp.int32),
                          (jnp.cumsum(nb * BM)[:-1]).astype(jnp.int32)])

    pos = (jnp.sum(jnp.where(oh, po[None, :], 0), axis=1)
           + rank).astype(jnp.int32)
    pos2 = pos.reshape(S, K)
    p0 = pos2[:, 0]
    p1 = pos2[:, 1]
    pos3 = pos2.reshape(NUM_WORKERS, TOK_PER_W, K).transpose(0, 2, 1)

    x_sorted, w_rows = _dispatch(hs_pack, pos3, w16)
    rows = _mlp(eob, x_sorted, _cast_w(gate_up_proj, 2 * INTER // 8),
                gate_up_proj_bias, _cast_w(down_proj, H // 2),
                down_proj_bias, w_rows)
    out = _combine(rows, p0, p1)
    return out.reshape(B, S, H)
